# baseline (speedup 1.0000x reference)
import jax
import jax.numpy as jnp
from jax.experimental import pallas as pl

NUM_AGENTS = 16384
NUM_ABS = 64
C = 2048
NBLK = NUM_AGENTS // C


def _rotl(x, r):
    return (x << jnp.uint32(r)) | (x >> jnp.uint32(32 - r))


def _threefry_bits(x1):
    ks0 = jnp.uint32(0)
    ks1 = jnp.uint32(42)
    ks2 = jnp.uint32(0x1BD11BDA ^ 42)

    def rounds(x0, x1, rots):
        for r in rots:
            x0 = x0 + x1
            x1 = _rotl(x1, r)
            x1 = x0 ^ x1
        return x0, x1

    x0 = x1
    x1 = x0 ^ _rotl(x1, 13)
    x0, x1 = rounds(x0, x1, (15, 26, 6))
    x0, x1 = x0 + ks1, x1 + ks2 + jnp.uint32(1)
    x0, x1 = rounds(x0, x1, (17, 29, 16, 24))
    x0, x1 = x0 + ks2, x1 + ks0 + jnp.uint32(2)
    x0, x1 = rounds(x0, x1, (13, 15, 26, 6))
    x0, x1 = x0 + ks0, x1 + ks1 + jnp.uint32(3)
    x0, x1 = rounds(x0, x1, (17, 29, 16, 24))
    x0, x1 = x0 + ks1, x1 + ks2 + jnp.uint32(4)
    x0, x1 = rounds(x0, x1, (13, 15, 26, 6))
    x0, x1 = x0 + ks2, x1 + ks0 + jnp.uint32(5)
    return x0 ^ x1


TILE = 256


def _body(logits_ref, out_ref):
    i = pl.program_id(0)
    b = pl.program_id(1)
    base = (b * (NUM_AGENTS * NUM_ABS) + i * (C * NUM_ABS)).astype(jnp.uint32)
    krow = jax.lax.broadcasted_iota(jnp.uint32, (NUM_ABS, TILE), 0)
    ncol = jax.lax.broadcasted_iota(jnp.uint32, (NUM_ABS, TILE), 1)
    cnt_lo = ncol * jnp.uint32(NUM_ABS) + krow
    kidx = krow.astype(jnp.int32)

    for j in range(C // TILE):
        col0 = j * TILE
        x1 = cnt_lo + (base + jnp.uint32(col0 * NUM_ABS + 42))
        bits = _threefry_bits(x1)
        m_bits = bits >> jnp.uint32(9)
        mf = jax.lax.bitcast_convert_type(m_bits, jnp.int32).astype(jnp.float32)
        s = jnp.log2(mf)
        q = (s - jnp.float32(23.0)) * logits_ref[:, pl.ds(col0, TILE)]

        m = jnp.max(q, axis=0, keepdims=True)
        idx = jnp.min(jnp.where(q == m, kidx, NUM_ABS), axis=0)
        out_ref[0, 0, pl.ds(col0, TILE)] = idx


def _run(batch):
    return pl.pallas_call(
        _body,
        grid=(NBLK, batch),
        in_specs=[pl.BlockSpec((NUM_ABS, C), lambda i, b: (0, i))],
        out_specs=pl.BlockSpec((1, 1, C), lambda i, b: (b * NBLK + i, 0, 0)),
        out_shape=jax.ShapeDtypeStruct((batch * NBLK, 1, C), jnp.int32),
    )


def kernel(state, assigner_logit_array):
    w_t = (jnp.float32(0.6931471805599453)
           * jnp.exp(-assigner_logit_array)).T
    if state.ndim == 2:
        batch = state.shape[0]
        return _run(batch)(w_t).reshape(batch, NUM_AGENTS)
    return _run(1)(w_t).reshape(NUM_AGENTS)


# speedup vs baseline: 1.0386x; 1.0386x over previous
import jax
import jax.numpy as jnp
from jax.experimental import pallas as pl

NUM_AGENTS = 16384
NUM_ABS = 64
C = 2048
NBLK = NUM_AGENTS // C


def _rotl(x, r):
    return (x << jnp.uint32(r)) | (x >> jnp.uint32(32 - r))


def _threefry_bits(x1):
    ks0 = jnp.uint32(0)
    ks1 = jnp.uint32(42)
    ks2 = jnp.uint32(0x1BD11BDA ^ 42)

    def rounds(x0, x1, rots):
        for r in rots:
            x0 = x0 + x1
            x1 = _rotl(x1, r)
            x1 = x0 ^ x1
        return x0, x1

    x0 = x1
    x1 = x0 ^ _rotl(x1, 13)
    x0, x1 = rounds(x0, x1, (15, 26, 6))
    x0, x1 = x0 + ks1, x1 + ks2 + jnp.uint32(1)
    x0, x1 = rounds(x0, x1, (17, 29, 16, 24))
    x0, x1 = x0 + ks2, x1 + ks0 + jnp.uint32(2)
    x0, x1 = rounds(x0, x1, (13, 15, 26, 6))
    x0, x1 = x0 + ks0, x1 + ks1 + jnp.uint32(3)
    x0, x1 = rounds(x0, x1, (17, 29, 16, 24))
    x0, x1 = x0 + ks1, x1 + ks2 + jnp.uint32(4)
    x0, x1 = rounds(x0, x1, (13, 15, 26, 6))
    x0, x1 = x0 + ks2, x1 + ks0 + jnp.uint32(5)
    return x0 ^ x1


TILE = 128


def _body(logits_ref, out_ref):
    i = pl.program_id(0)
    b = pl.program_id(1)
    base = (b * (NUM_AGENTS * NUM_ABS) + i * (C * NUM_ABS)).astype(jnp.uint32)
    krow = jax.lax.broadcasted_iota(jnp.uint32, (NUM_ABS, TILE), 0)
    ncol = jax.lax.broadcasted_iota(jnp.uint32, (NUM_ABS, TILE), 1)
    cnt_lo = ncol * jnp.uint32(NUM_ABS) + krow
    kidx = krow.astype(jnp.int32)

    for j in range(C // TILE):
        col0 = j * TILE
        x1 = cnt_lo + (base + jnp.uint32(col0 * NUM_ABS + 42))
        bits = _threefry_bits(x1)
        m_bits = bits >> jnp.uint32(9)
        mf = jax.lax.bitcast_convert_type(m_bits, jnp.int32).astype(jnp.float32)
        s = jnp.log(mf)
        q = (s - jnp.float32(15.942385152878742)) * logits_ref[:, pl.ds(col0, TILE)]

        m = jnp.max(q, axis=0, keepdims=True)
        idx = jnp.min(jnp.where(q == m, kidx, NUM_ABS), axis=0)
        out_ref[0, 0, pl.ds(col0, TILE)] = idx


def _run(batch):
    return pl.pallas_call(
        _body,
        grid=(NBLK, batch),
        in_specs=[pl.BlockSpec((NUM_ABS, C), lambda i, b: (0, i))],
        out_specs=pl.BlockSpec((1, 1, C), lambda i, b: (b * NBLK + i, 0, 0)),
        out_shape=jax.ShapeDtypeStruct((batch * NBLK, 1, C), jnp.int32),
    )


def kernel(state, assigner_logit_array):
    w_t = jnp.exp(-assigner_logit_array).T
    if state.ndim == 2:
        batch = state.shape[0]
        return _run(batch)(w_t).reshape(batch, NUM_AGENTS)
    return _run(1)(w_t).reshape(NUM_AGENTS)


# speedup vs baseline: 1.0787x; 1.0385x over previous
import jax
import jax.numpy as jnp
from jax.experimental import pallas as pl

NUM_AGENTS = 16384
NUM_ABS = 64
C = 16384
NBLK = NUM_AGENTS // C


def _rotl(x, r):
    return (x << jnp.uint32(r)) | (x >> jnp.uint32(32 - r))


def _threefry_bits(x1):
    ks0 = jnp.uint32(0)
    ks1 = jnp.uint32(42)
    ks2 = jnp.uint32(0x1BD11BDA ^ 42)

    def rounds(x0, x1, rots):
        for r in rots:
            x0 = x0 + x1
            x1 = _rotl(x1, r)
            x1 = x0 ^ x1
        return x0, x1

    x0 = x1
    x1 = x0 ^ _rotl(x1, 13)
    x0, x1 = rounds(x0, x1, (15, 26, 6))
    x0, x1 = x0 + ks1, x1 + ks2 + jnp.uint32(1)
    x0, x1 = rounds(x0, x1, (17, 29, 16, 24))
    x0, x1 = x0 + ks2, x1 + ks0 + jnp.uint32(2)
    x0, x1 = rounds(x0, x1, (13, 15, 26, 6))
    x0, x1 = x0 + ks0, x1 + ks1 + jnp.uint32(3)
    x0, x1 = rounds(x0, x1, (17, 29, 16, 24))
    x0, x1 = x0 + ks1, x1 + ks2 + jnp.uint32(4)
    x0, x1 = rounds(x0, x1, (13, 15, 26, 6))
    x0, x1 = x0 + ks2, x1 + ks0 + jnp.uint32(5)
    return x0 ^ x1


TILE = 128


def _body(logits_ref, out_ref):
    i = pl.program_id(0)
    b = pl.program_id(1)
    base = (b * (NUM_AGENTS * NUM_ABS) + i * (C * NUM_ABS)).astype(jnp.uint32)
    krow = jax.lax.broadcasted_iota(jnp.uint32, (NUM_ABS, TILE), 0)
    ncol = jax.lax.broadcasted_iota(jnp.uint32, (NUM_ABS, TILE), 1)
    cnt_lo = ncol * jnp.uint32(NUM_ABS) + krow
    kpack = krow | jnp.uint32(0x80000000)

    for j in range(C // TILE):
        col0 = j * TILE
        x1 = cnt_lo + (base + jnp.uint32(col0 * NUM_ABS + 42))
        bits = _threefry_bits(x1)
        m_bits = bits >> jnp.uint32(9)
        mf = jax.lax.bitcast_convert_type(m_bits, jnp.int32).astype(jnp.float32)
        s = jnp.log(mf)
        q = (s - jnp.float32(15.942385152878742)) * logits_ref[:, pl.ds(col0, TILE)]

        qb = jax.lax.bitcast_convert_type(q, jnp.uint32)
        key = jax.lax.bitcast_convert_type(
            (qb & jnp.uint32(0xFFFFFFC0)) | kpack, jnp.int32)
        idx = jnp.min(key, axis=0) & jnp.int32(63)
        out_ref[0, 0, pl.ds(col0, TILE)] = idx


def _run(batch):
    return pl.pallas_call(
        _body,
        grid=(NBLK, batch),
        in_specs=[pl.BlockSpec((NUM_ABS, C), lambda i, b: (0, i))],
        out_specs=pl.BlockSpec((1, 1, C), lambda i, b: (b * NBLK + i, 0, 0)),
        out_shape=jax.ShapeDtypeStruct((batch * NBLK, 1, C), jnp.int32),
    )


def kernel(state, assigner_logit_array):
    w_t = jnp.exp(-assigner_logit_array).T
    if state.ndim == 2:
        batch = state.shape[0]
        return _run(batch)(w_t).reshape(batch, NUM_AGENTS)
    return _run(1)(w_t).reshape(NUM_AGENTS)


# speedup vs baseline: 1.1055x; 1.0249x over previous
import jax
import jax.numpy as jnp
from jax.experimental import pallas as pl

NUM_AGENTS = 16384
NUM_ABS = 64
C = 16384
NBLK = NUM_AGENTS // C


def _rotl(x, r):
    return (x << jnp.uint32(r)) | (x >> jnp.uint32(32 - r))


def _threefry_bits(x1):
    ks1 = jnp.uint32(42)
    ks2 = jnp.uint32(0x1BD11BDA ^ 42)

    def rounds(x0, x1, rots):
        for r in rots:
            x0 = x0 + x1
            x1 = _rotl(x1, r)
            x1 = x0 ^ x1
        return x0, x1

    x0 = x1
    x1 = x0 ^ _rotl(x1, 13)
    x0, x1 = rounds(x0, x1, (15, 26, 6))
    x0, x1 = x0 + ks1, x1 + jnp.uint32((0x1BD11BDA ^ 42) + 1)
    x0, x1 = rounds(x0, x1, (17, 29, 16, 24))
    x0, x1 = x0 + ks2, x1 + jnp.uint32(2)
    x0, x1 = rounds(x0, x1, (13, 15, 26, 6))
    x0, x1 = x0, x1 + jnp.uint32(42 + 3)
    x0, x1 = rounds(x0, x1, (17, 29, 16, 24))
    x0, x1 = x0 + ks1, x1 + jnp.uint32(((0x1BD11BDA ^ 42) + 4) & 0xFFFFFFFF)
    x0, x1 = rounds(x0, x1, (13, 15, 26, 6))
    x0, x1 = x0 + ks2, x1 + jnp.uint32(5)
    return x0 ^ x1


TILE = 128


def _body(iota_ref, logits_ref, out_ref):
    i = pl.program_id(0)
    b = pl.program_id(1)
    base = (b * (NUM_AGENTS * NUM_ABS) + i * (C * NUM_ABS)).astype(jnp.uint32)
    krow = jax.lax.broadcasted_iota(jnp.uint32, (NUM_ABS, TILE), 0)
    kpack = krow | jnp.uint32(0x80000000)

    for j in range(C // TILE):
        col0 = j * TILE
        x1 = iota_ref[...] + (base + jnp.uint32(col0 * NUM_ABS + 42))
        bits = _threefry_bits(x1)
        m_bits = bits >> jnp.uint32(9)
        mf = jax.lax.bitcast_convert_type(m_bits, jnp.int32).astype(jnp.float32)
        s = jnp.log(mf)
        q = (s - jnp.float32(15.942385152878742)) * logits_ref[:, pl.ds(col0, TILE)]

        qb = jax.lax.bitcast_convert_type(q, jnp.uint32)
        key = jax.lax.bitcast_convert_type(
            (qb & jnp.uint32(0xFFFFFFC0)) | kpack, jnp.int32)
        idx = jnp.min(key, axis=0) & jnp.int32(63)
        out_ref[0, 0, pl.ds(col0, TILE)] = idx


def _run(batch):
    return pl.pallas_call(
        _body,
        grid=(NBLK, batch),
        in_specs=[pl.BlockSpec((NUM_ABS, TILE), lambda i, b: (0, 0)),
                  pl.BlockSpec((NUM_ABS, C), lambda i, b: (0, i))],
        out_specs=pl.BlockSpec((1, 1, C), lambda i, b: (b * NBLK + i, 0, 0)),
        out_shape=jax.ShapeDtypeStruct((batch * NBLK, 1, C), jnp.int32),
    )


def kernel(state, assigner_logit_array):
    w_t = jnp.exp(-assigner_logit_array).T
    iota = (jnp.arange(TILE, dtype=jnp.uint32)[None, :] * jnp.uint32(NUM_ABS)
            + jnp.arange(NUM_ABS, dtype=jnp.uint32)[:, None])
    if state.ndim == 2:
        batch = state.shape[0]
        return _run(batch)(iota, w_t).reshape(batch, NUM_AGENTS)
    return _run(1)(iota, w_t).reshape(NUM_AGENTS)
